# Initial kernel scaffold; baseline (speedup 1.0000x reference)
import jax
import jax.numpy as jnp
from jax.experimental import pallas as pl


def kernel(inputs, targets):
    raise NotImplementedError("write your pallas kernel here")


# speedup vs baseline: 1.4743x; 1.4743x over previous
import functools

import jax
import jax.numpy as jnp
import numpy as np
from jax.experimental import pallas as pl
from jax.experimental.pallas import tpu as pltpu

_HARD_RATIO = 0.25
_MIN_HARD_NUM = 4


def _ohem_kernel(x_ref, t_ref, o_ref, loss_ref, *, nb, k):
    i = pl.program_id(0)
    x = x_ref[...]
    t = t_ref[0]
    col = jax.lax.broadcasted_iota(jnp.int32, x.shape, 1)
    m = jnp.max(x, axis=1, keepdims=True)
    s = jnp.sum(jnp.exp(x - m), axis=1, keepdims=True)
    tl = jnp.sum(jnp.where(col == t, x, 0.0), axis=1, keepdims=True)
    loss = m + jnp.log(s) - tl
    rb = x.shape[0]
    nr = rb // 128
    loss_ref[pl.ds(i * nr, nr), :] = loss.reshape(nr, 128)

    @pl.when(i == nb - 1)
    def _select():
        vals = loss_ref[...]
        bits = jax.lax.bitcast_convert_type(vals, jnp.uint32)
        flip = jnp.where((bits >> 31) == jnp.uint32(1),
                         jnp.uint32(0xFFFFFFFF), jnp.uint32(0x80000000))
        key = bits ^ flip

        def body(_, carry):
            T, bit = carry
            cand = T | bit
            cnt = jnp.sum((key >= cand).astype(jnp.int32))
            return (jax.lax.select(cnt >= k, cand, T), bit >> 1)

        (T, _b) = jax.lax.fori_loop(
            0, 32, body, (jnp.uint32(0), jnp.uint32(0x80000000)))
        gt = key > T
        cnt_gt = jnp.sum(gt.astype(jnp.int32))
        sum_gt = jnp.sum(jnp.where(gt, vals, 0.0))
        tval = jnp.min(jnp.where(key >= T, vals, jnp.float32(np.inf)))
        res = (sum_gt
               + (k - cnt_gt).astype(jnp.float32) * tval) / jnp.float32(k)
        o_ref[...] = res.reshape(1, 1)


def kernel(inputs, targets):
    batch, classes = inputs.shape
    k = max(int(batch * _HARD_RATIO), min(_MIN_HARD_NUM, batch))
    k = min(k, batch)
    rb = 1024
    nb = batch // rb
    t3 = targets.astype(jnp.int32).reshape(nb, rb, 1)
    out = pl.pallas_call(
        functools.partial(_ohem_kernel, nb=nb, k=k),
        grid=(nb,),
        in_specs=[
            pl.BlockSpec((rb, classes), lambda i: (i, 0)),
            pl.BlockSpec((1, rb, 1), lambda i: (i, 0, 0)),
        ],
        out_specs=pl.BlockSpec((1, 1), lambda i: (0, 0)),
        out_shape=jax.ShapeDtypeStruct((1, 1), jnp.float32),
        scratch_shapes=[pltpu.VMEM((batch // 128, 128), jnp.float32)],
    )(inputs, t3)
    return out[0, 0]
